# baseline (speedup 1.0000x reference)
import functools

import jax
import jax.numpy as jnp
from jax import lax
from jax.experimental import pallas as pl
from jax.experimental.pallas import tpu as pltpu
from jax.experimental.pallas import tpu_sc as plsc

N_NODES = 10000
N_EDGES = 320000
IN_FEATS = 128
N_CLASSES = 64
K_HOPS = 3

NCORE = 2
NSUB = 16
NBLK = 4
BLKW = N_CLASSES // NBLK

CHUNK = 128
NCHUNK = 160
DEPTH = 8
NGEN = NCHUNK // DEPTH
EDGES_PER_TILE = NCHUNK * CHUNK
EDGES_PAD = NSUB * EDGES_PER_TILE
NPAD = 10112
ROWS_PER_TILE = NPAD // NSUB


def _proj_body(feat_ref, w_ref, out_ref):
    h = lax.dot_general(
        feat_ref[...], w_ref[0],
        (((1,), (1,)), ((), ())),
        preferred_element_type=jnp.float32,
    )
    out_ref[0] = jnp.concatenate(
        [h, jnp.zeros((NPAD - N_NODES, BLKW), jnp.float32)], axis=0
    )


def _project(feat, w_split):
    return pl.pallas_call(
        _proj_body,
        grid=(NBLK,),
        in_specs=[
            pl.BlockSpec((N_NODES, IN_FEATS), lambda i: (0, 0)),
            pl.BlockSpec((1, BLKW, IN_FEATS), lambda i: (i, 0, 0)),
        ],
        out_specs=pl.BlockSpec((1, NPAD, BLKW), lambda i: (i, 0, 0)),
        out_shape=jax.ShapeDtypeStruct((NBLK, NPAD, BLKW), jnp.float32),
    )(feat, w_split)


def _spmm_body(p_hbm, srcs_hbm, dsts_hbm, zz_hbm, bb_hbm,
               out_hbm, wka_hbm, wkb_hbm, sidx, didx, buf_a, buf_b, hs, acc,
               gsem_a, gsem_b, ssem_a, ssem_b):
    c = lax.axis_index("c")
    s = lax.axis_index("s")
    row0 = s * ROWS_PER_TILE
    slab = pl.ds(row0, ROWS_PER_TILE)

    pltpu.sync_copy(srcs_hbm.at[s], sidx)
    pltpu.sync_copy(dsts_hbm.at[s], didx)

    def fire_gathers(g, buf, sem):
        for d in range(DEPTH):
            pltpu.async_copy(hs.at[sidx.at[g * DEPTH + d]], buf.at[d], sem)

    def drain_gathers(buf, sem):
        for d in range(DEPTH):
            pltpu.make_async_copy(hs.at[sidx.at[0]], buf.at[d], sem).wait()

    def fire_scatters(g, buf, sem):
        for d in range(DEPTH):
            pltpu.async_copy(buf.at[d], acc.at[didx.at[g * DEPTH + d]],
                             sem, add=True)

    def drain_scatters(buf, sem):
        for d in range(DEPTH):
            pltpu.make_async_copy(buf.at[d], acc.at[didx.at[0]], sem).wait()

    for h in range(K_HOPS):
        h_in = (p_hbm, wka_hbm, wkb_hbm)[h]
        h_out = (wka_hbm, wkb_hbm, out_hbm)[h]
        for p_idx in range(2):
            q = 2 * c + p_idx
            pltpu.sync_copy(h_in.at[q, slab], hs.at[slab])
            if h == K_HOPS - 1:
                pltpu.sync_copy(bb_hbm.at[q], acc.at[slab])
            else:
                pltpu.sync_copy(zz_hbm, acc.at[slab])
            plsc.subcore_barrier()

            fire_gathers(0, buf_a, gsem_a)

            @pl.loop(0, NGEN, step=2)
            def _(g):
                drain_gathers(buf_a, gsem_a)

                @pl.when(g > 0)
                def _():
                    drain_scatters(buf_b, ssem_b)
                fire_gathers(g + 1, buf_b, gsem_b)
                fire_scatters(g, buf_a, ssem_a)

                drain_gathers(buf_b, gsem_b)
                drain_scatters(buf_a, ssem_a)

                @pl.when(g + 2 < NGEN)
                def _():
                    fire_gathers(g + 2, buf_a, gsem_a)
                fire_scatters(g + 1, buf_b, ssem_b)

            drain_scatters(buf_b, ssem_b)
            plsc.subcore_barrier()

            pltpu.sync_copy(acc.at[slab], h_out.at[q, slab])
            plsc.subcore_barrier()


@functools.lru_cache(maxsize=None)
def _make_spmm():
    return pl.kernel(
        _spmm_body,
        out_type=(
            jax.ShapeDtypeStruct((NBLK, NPAD, BLKW), jnp.float32),
            jax.ShapeDtypeStruct((NBLK, NPAD, BLKW), jnp.float32),
            jax.ShapeDtypeStruct((NBLK, NPAD, BLKW), jnp.float32),
        ),
        mesh=plsc.VectorSubcoreMesh(core_axis_name="c", subcore_axis_name="s",
                                    num_cores=NCORE, num_subcores=NSUB),
        scratch_types=[
            pltpu.VMEM((NCHUNK, CHUNK), jnp.int32),
            pltpu.VMEM((NCHUNK, CHUNK), jnp.int32),
            pltpu.VMEM((DEPTH, CHUNK, BLKW), jnp.float32),
            pltpu.VMEM((DEPTH, CHUNK, BLKW), jnp.float32),
            pltpu.VMEM_SHARED((NPAD, BLKW), jnp.float32),
            pltpu.VMEM_SHARED((NPAD, BLKW), jnp.float32),
            pltpu.SemaphoreType.DMA,
            pltpu.SemaphoreType.DMA,
            pltpu.SemaphoreType.DMA,
            pltpu.SemaphoreType.DMA,
        ],
        compiler_params=pltpu.CompilerParams(use_tc_tiling_on_sc=False),
    )


def kernel(feat, edge_index, W, b):
    src = edge_index[0].astype(jnp.int32)
    dst = edge_index[1].astype(jnp.int32)
    pad = EDGES_PAD - N_EDGES
    src_p = jnp.concatenate([src, jnp.zeros((pad,), jnp.int32)])
    dst_p = jnp.concatenate([dst, jnp.full((pad,), N_NODES, jnp.int32)])
    srcs = src_p.reshape(NSUB, NCHUNK, CHUNK)
    dsts = dst_p.reshape(NSUB, NCHUNK, CHUNK)
    zz = jnp.zeros((ROWS_PER_TILE, BLKW), jnp.float32)
    bb = jnp.broadcast_to(
        b.reshape(NBLK, 1, BLKW), (NBLK, ROWS_PER_TILE, BLKW))

    p = _project(feat, W.reshape(NBLK, BLKW, IN_FEATS))
    res, _, _ = _make_spmm()(p, srcs, dsts, zz, bb)
    return res[:, :N_NODES, :].transpose(1, 0, 2).reshape(N_NODES, N_CLASSES)
